# baseline (speedup 1.0000x reference)
import jax
import jax.numpy as jnp
from jax import lax
from jax.experimental import pallas as pl
from jax.experimental.pallas import tpu as pltpu, tpu_sc as plsc

N_NODES = 10000
N_PAD = 10240
ROWS_PER_TILE = 640
N_EDGES = 160000
EDGES_PER_TILE = 10240
NCHUNK = 80
C = 128
N_IN = 128
N_OUT = 128
BATCH = 128
HALF = 64
ITERS = 20
LEAK = 0.01

_mesh = plsc.VectorSubcoreMesh(
    core_axis_name="c", subcore_axis_name="s", num_cores=2, num_subcores=16)


def _mml_vec(v):
    v = jnp.where(v < 0, v * LEAK, v)
    safe = jnp.where(v > 0.5, v, 1.0)
    return jnp.where(v > 0.5, 1.0 - 0.25 / safe, v)


def _body(xt_h, w_in_h, biases_h, w_out_h, in_idx_h, out_idx_h,
          cols_h, rows_h, ew_h,
          o_h, xhat_h, bin_h,
          cols_v, rows_v, ew_v, gbuf, gbuf1, gbuf2,
          wiv, wov, iiv, oiv, bv, sem, sg, ss, sx, sr, ax_sh):
    c = lax.axis_index("c")
    s = lax.axis_index("s")
    base = s * ROWS_PER_TILE
    hoff = c * N_PAD

    pltpu.sync_copy(cols_h.at[s], cols_v)
    pltpu.sync_copy(rows_h.at[s], rows_v)
    pltpu.sync_copy(ew_h.at[s], ew_v)
    pltpu.sync_copy(biases_h.at[pl.ds(base, ROWS_PER_TILE)], bv)
    pltpu.sync_copy(out_idx_h, oiv)
    pltpu.sync_copy(w_out_h, wov.at[pl.ds(0, N_OUT)])

    @pl.loop(0, NCHUNK)
    def _shift(ch):
        for g in range(8):
            sl = pl.ds(g * 16, 16)
            cols_v[ch, sl] = cols_v[ch, sl] + hoff
    @pl.loop(0, N_OUT // 16)
    def _shift_o(g):
        sl = pl.ds(g * 16, 16)
        oiv[sl] = oiv[sl] + hoff

    @pl.loop(0, ROWS_PER_TILE // 16)
    def _bias_row(g):
        bvv = bv[pl.ds(g * 16, 16)]
        for j in range(16):
            row = jnp.full((16,), bvv[j])
            for k in range(4):
                gbuf2[j, pl.ds(k * 16, 16)] = row
        pltpu.sync_copy(gbuf2.at[pl.ds(0, 16)],
                        ax_sh.at[pl.ds(base + g * 16, 16)])

    plsc.subcore_barrier()

    @pl.when(s == 0)
    def _input_scatter():
        pltpu.sync_copy(w_in_h, wiv)
        pltpu.sync_copy(in_idx_h, iiv)
        pltpu.sync_copy(xt_h.at[c], gbuf.at[pl.ds(0, C)])

        @pl.loop(0, N_IN // 16)
        def _scale_in(g):
            wvv = wiv[pl.ds(g * 16, 16)]
            for j in range(16):
                wrow = jnp.full((16,), wvv[j])
                r = g * 16 + j
                for k in range(4):
                    sl = pl.ds(k * 16, 16)
                    gbuf[r, sl] = gbuf[r, sl] * wrow

        pltpu.sync_copy(gbuf.at[pl.ds(0, C)], ax_sh.at[iiv], add=True)

    plsc.subcore_barrier()

    @pl.loop(0, ROWS_PER_TILE // C)
    def _save_bin(i):
        off = base + i * C
        pltpu.sync_copy(ax_sh.at[pl.ds(off, C)], gbuf2.at[pl.ds(C, C)])
        pltpu.sync_copy(gbuf2.at[pl.ds(C, C)], bin_h.at[pl.ds(hoff + off, C)])

    def _mml_buf():
        @plsc.parallel_loop(0, C, unroll=2)
        def _act_row(r):
            for k in range(4):
                sl = pl.ds(k * 16, 16)
                gbuf2[r, sl] = _mml_vec(gbuf2[r, sl])

    def act_phase():
        bin_cp = pltpu.async_copy(bin_h.at[pl.ds(hoff + base, C)], gbuf2.at[pl.ds(C, C)], sem)
        pltpu.sync_copy(ax_sh.at[pl.ds(base, C)], gbuf2.at[pl.ds(0, C)])
        _mml_buf()
        pltpu.async_copy(gbuf2.at[pl.ds(0, C)], xhat_h.at[pl.ds(hoff + base, C)], sx)
        bin_cp.wait()
        pltpu.async_copy(gbuf2.at[pl.ds(C, C)], ax_sh.at[pl.ds(base, C)], sr)

        @pl.loop(1, ROWS_PER_TILE // C)
        def _act_sub(i):
            off = base + i * C
            pltpu.make_async_copy(gbuf2.at[pl.ds(C, C)], ax_sh.at[pl.ds(base, C)], sr).wait()
            bin_cp = pltpu.async_copy(
                bin_h.at[pl.ds(hoff + off, C)], gbuf2.at[pl.ds(C, C)], sem)
            pltpu.make_async_copy(
                gbuf2.at[pl.ds(0, C)], xhat_h.at[pl.ds(hoff + base, C)],
                sx).wait()
            pltpu.sync_copy(ax_sh.at[pl.ds(off, C)], gbuf2.at[pl.ds(0, C)])
            _mml_buf()
            pltpu.async_copy(gbuf2.at[pl.ds(0, C)], xhat_h.at[pl.ds(hoff + off, C)], sx)
            bin_cp.wait()
            pltpu.async_copy(gbuf2.at[pl.ds(C, C)], ax_sh.at[pl.ds(off, C)], sr)

        pltpu.make_async_copy(gbuf2.at[pl.ds(C, C)], ax_sh.at[pl.ds(base, C)], sr).wait()
        pltpu.make_async_copy(
            gbuf2.at[pl.ds(0, C)], xhat_h.at[pl.ds(hoff + base, C)],
            sx).wait()

    def scale2(buf, j2):
        @plsc.parallel_loop(0, 2 * C // 16, unroll=4)
        def _scale(g):
            ch = 2 * j2 + g // 8
            wvv = ew_v[ch, pl.ds((g % 8) * 16, 16)]
            for jj in range(16):
                wrow = jnp.full((16,), wvv[jj])
                e = g * 16 + jj
                for k in range(4):
                    sl = pl.ds(k * 16, 16)
                    buf[e, sl] = buf[e, sl] * wrow

    def start_gather2(j2, buf):
        pltpu.async_copy(xhat_h.at[cols_v.at[2 * j2]], buf.at[pl.ds(0, C)], sg)
        pltpu.async_copy(xhat_h.at[cols_v.at[2 * j2 + 1]],
                         buf.at[pl.ds(C, C)], sg)

    def wait_gather2(buf):
        pltpu.make_async_copy(xhat_h.at[pl.ds(0, 2 * C)], buf, sg).wait()

    def start_scatter2(j2, buf):
        pltpu.async_copy(buf.at[pl.ds(0, C)], ax_sh.at[rows_v.at[2 * j2]],
                         ss, add=True)
        pltpu.async_copy(buf.at[pl.ds(C, C)], ax_sh.at[rows_v.at[2 * j2 + 1]],
                         ss, add=True)

    def wait_scatter_any():
        pltpu.make_async_copy(gbuf, ax_sh.at[pl.ds(0, 2 * C)], ss).wait()

    NSC = NCHUNK // 2
    bufs = (gbuf, gbuf1, gbuf2)

    def roundu(j, k):
        b = bufs[k]
        wait_gather2(b)
        scale2(b, j)
        start_scatter2(j, b)
        wait_scatter_any()
        start_gather2(j + 2, bufs[(k + 2) % 3])

    def edge_phase():
        start_gather2(0, gbuf)
        start_gather2(1, gbuf1)
        wait_gather2(gbuf)
        scale2(gbuf, 0)
        start_scatter2(0, gbuf)
        start_gather2(2, gbuf2)
        roundu(1, 1)
        roundu(2, 2)

        @pl.loop(0, (NSC - 6) // 3)
        def _triple(t):
            j = 3 * t + 3
            roundu(j, 0)
            roundu(j + 1, 1)
            roundu(j + 2, 2)

        roundu(NSC - 4, 0)
        roundu(NSC - 3, 1)
        wait_gather2(gbuf2)
        scale2(gbuf2, NSC - 2)
        start_scatter2(NSC - 2, gbuf2)
        wait_gather2(gbuf)
        scale2(gbuf, NSC - 1)
        start_scatter2(NSC - 1, gbuf)
        wait_scatter_any()
        wait_scatter_any()
        wait_scatter_any()

    act_phase()
    plsc.subcore_barrier()

    @pl.loop(0, ITERS - 1)
    def _iter(it):
        edge_phase()
        plsc.subcore_barrier()
        act_phase()
        plsc.subcore_barrier()

    pltpu.async_copy(xhat_h.at[oiv.at[pl.ds(s * 8, 8)]],
                     gbuf.at[pl.ds(0, 8)], sem).wait()
    wv16 = wov[pl.ds(s * 8, 16)]
    for j in range(8):
        wrow = jnp.full((16,), wv16[j])
        for k in range(4):
            sl = pl.ds(k * 16, 16)
            gbuf[j, sl] = gbuf[j, sl] * wrow
    pltpu.sync_copy(gbuf.at[pl.ds(0, 8)], o_h.at[c, pl.ds(s * 8, 8)])


_sc_call = pl.kernel(
    _body,
    out_type=(
        jax.ShapeDtypeStruct((2, N_OUT, HALF), jnp.float32),
        jax.ShapeDtypeStruct((2 * N_PAD, HALF), jnp.float32),
        jax.ShapeDtypeStruct((2 * N_PAD, HALF), jnp.float32),
    ),
    mesh=_mesh,
    compiler_params=pltpu.CompilerParams(use_tc_tiling_on_sc=False),
    scratch_types=[
        pltpu.VMEM((NCHUNK, C), jnp.int32),
        pltpu.VMEM((NCHUNK, C), jnp.int32),
        pltpu.VMEM((NCHUNK, C), jnp.float32),
        pltpu.VMEM((2 * C, HALF), jnp.float32),
        pltpu.VMEM((2 * C, HALF), jnp.float32),
        pltpu.VMEM((2 * C, HALF), jnp.float32),
        pltpu.VMEM((N_IN,), jnp.float32),
        pltpu.VMEM((N_OUT + 16,), jnp.float32),
        pltpu.VMEM((N_IN,), jnp.int32),
        pltpu.VMEM((N_OUT,), jnp.int32),
        pltpu.VMEM((ROWS_PER_TILE,), jnp.float32),
        pltpu.SemaphoreType.DMA,
        pltpu.SemaphoreType.DMA,
        pltpu.SemaphoreType.DMA,
        pltpu.SemaphoreType.DMA,
        pltpu.SemaphoreType.DMA,
        pltpu.VMEM_SHARED((N_PAD, HALF), jnp.float32),
    ],
)


def kernel(x, w_in, edge_w, biases, w_out, rows, cols, in_idx, out_idx):
    xt2 = x.T.reshape(N_IN, 2, HALF).transpose(1, 0, 2)
    biases_p = jnp.pad(biases.reshape(-1), (0, N_PAD - N_NODES))
    n_pad_e = 16 * EDGES_PER_TILE - N_EDGES
    pad_idx = ((jnp.arange(n_pad_e, dtype=jnp.int32) * 37) % N_NODES)
    cols_p = jnp.concatenate([cols, pad_idx]).reshape(16, NCHUNK, C)
    rows_p = jnp.concatenate([rows, pad_idx]).reshape(16, NCHUNK, C)
    ew_p = jnp.concatenate(
        [edge_w, jnp.zeros((n_pad_e,), jnp.float32)]).reshape(16, NCHUNK, C)

    o, _, _ = _sc_call(xt2, w_in, biases_p, w_out,
                       in_idx.astype(jnp.int32), out_idx.astype(jnp.int32),
                       cols_p, rows_p, ew_p)
    return jnp.concatenate([o[0], o[1]], axis=1).T


# speedup vs baseline: 1.0726x; 1.0726x over previous
import jax
import jax.numpy as jnp
from jax import lax
from jax.experimental import pallas as pl
from jax.experimental.pallas import tpu as pltpu, tpu_sc as plsc

N_NODES = 10000
N_PAD = 10240
ROWS_PER_TILE = 640
N_EDGES = 160000
EDGES_PER_TILE = 10240
NCHUNK = 80
C = 128
N_IN = 128
N_OUT = 128
BATCH = 128
HALF = 64
ITERS = 20
LEAK = 0.01

_mesh = plsc.VectorSubcoreMesh(
    core_axis_name="c", subcore_axis_name="s", num_cores=2, num_subcores=16)

_PK = None


def _mml_vec(v):
    v = jnp.where(v < 0, v * LEAK, v)
    safe = jnp.where(v > 0.5, v, 1.0)
    return jnp.where(v > 0.5, 1.0 - 0.25 / safe, v)


def _body(xt_h, w_in_h, biases_h, w_out_h, in_idx_h, out_idx_h,
          cols_h, rows_h, ew_h,
          o_h, xhat_h, bin_h,
          cols_v, rows_v, ew_v, gbuf, gbuf1, gb16a, gb16b, hbuf,
          wiv, wov, iiv, oiv, bv, sem, sg, ss, sx, sr, ax_sh):
    fmt = plsc.PackFormat.INTERLEAVED
    c = lax.axis_index("c")
    s = lax.axis_index("s")
    base = s * ROWS_PER_TILE
    hoff = c * N_PAD

    pltpu.sync_copy(cols_h.at[s], cols_v)
    pltpu.sync_copy(rows_h.at[s], rows_v)
    pltpu.sync_copy(ew_h.at[s], ew_v)
    pltpu.sync_copy(biases_h.at[pl.ds(base, ROWS_PER_TILE)], bv)
    pltpu.sync_copy(out_idx_h, oiv)
    pltpu.sync_copy(w_out_h, wov.at[pl.ds(0, N_OUT)])

    @pl.loop(0, NCHUNK)
    def _shift(ch):
        for g in range(8):
            sl = pl.ds(g * 16, 16)
            cols_v[ch, sl] = cols_v[ch, sl] + hoff

    @pl.loop(0, N_OUT // 16)
    def _shift_o(g):
        sl = pl.ds(g * 16, 16)
        oiv[sl] = oiv[sl] + hoff

    @pl.loop(0, ROWS_PER_TILE // 16)
    def _bias_row(g):
        bvv = bv[pl.ds(g * 16, 16)]
        for j in range(16):
            row = jnp.full((16,), bvv[j])
            for k in range(4):
                gbuf1[j, pl.ds(k * 16, 16)] = row
        pltpu.sync_copy(gbuf1.at[pl.ds(0, 16)],
                        ax_sh.at[pl.ds(base + g * 16, 16)])

    plsc.subcore_barrier()

    @pl.when(s == 0)
    def _input_scatter():
        pltpu.sync_copy(w_in_h, wiv)
        pltpu.sync_copy(in_idx_h, iiv)
        pltpu.sync_copy(xt_h.at[c], gbuf.at[pl.ds(0, C)])

        @pl.loop(0, N_IN // 16)
        def _scale_in(g):
            wvv = wiv[pl.ds(g * 16, 16)]
            for j in range(16):
                wrow = jnp.full((16,), wvv[j])
                r = g * 16 + j
                for k in range(4):
                    sl = pl.ds(k * 16, 16)
                    gbuf[r, sl] = gbuf[r, sl] * wrow

        pltpu.sync_copy(gbuf.at[pl.ds(0, C)], ax_sh.at[iiv], add=True)

    plsc.subcore_barrier()

    @pl.loop(0, ROWS_PER_TILE // C)
    def _save_bin(i):
        off = base + i * C
        pltpu.sync_copy(ax_sh.at[pl.ds(off, C)], gbuf1.at[pl.ds(C, C)])
        pltpu.sync_copy(gbuf1.at[pl.ds(C, C)], bin_h.at[pl.ds(hoff + off, C)])

    def _mml_pack_buf():
        @plsc.parallel_loop(0, C, unroll=2)
        def _act_row(r):
            v0 = _mml_vec(gbuf1[r, pl.ds(0, 16)])
            v1 = _mml_vec(gbuf1[r, pl.ds(16, 16)])
            v2 = _mml_vec(gbuf1[r, pl.ds(32, 16)])
            v3 = _mml_vec(gbuf1[r, pl.ds(48, 16)])
            hbuf[r, pl.ds(0, 32)] = plsc.pack(v0, v1, format=fmt)
            hbuf[r, pl.ds(32, 32)] = plsc.pack(v2, v3, format=fmt)

    def act_phase():
        bin_cp = pltpu.async_copy(
            bin_h.at[pl.ds(hoff + base, C)], gbuf1.at[pl.ds(C, C)], sem)
        pltpu.sync_copy(ax_sh.at[pl.ds(base, C)], gbuf1.at[pl.ds(0, C)])
        _mml_pack_buf()
        pltpu.async_copy(hbuf, xhat_h.at[pl.ds(hoff + base, C)], sx)
        bin_cp.wait()
        pltpu.async_copy(gbuf1.at[pl.ds(C, C)], ax_sh.at[pl.ds(base, C)], sr)

        @pl.loop(1, ROWS_PER_TILE // C)
        def _act_sub(i):
            off = base + i * C
            pltpu.make_async_copy(
                gbuf1.at[pl.ds(C, C)], ax_sh.at[pl.ds(base, C)], sr).wait()
            bin_cp = pltpu.async_copy(
                bin_h.at[pl.ds(hoff + off, C)], gbuf1.at[pl.ds(C, C)], sem)
            pltpu.sync_copy(ax_sh.at[pl.ds(off, C)], gbuf1.at[pl.ds(0, C)])
            pltpu.make_async_copy(
                hbuf, xhat_h.at[pl.ds(hoff + base, C)], sx).wait()
            _mml_pack_buf()
            pltpu.async_copy(hbuf, xhat_h.at[pl.ds(hoff + off, C)], sx)
            bin_cp.wait()
            pltpu.async_copy(
                gbuf1.at[pl.ds(C, C)], ax_sh.at[pl.ds(off, C)], sr)

        pltpu.make_async_copy(
            gbuf1.at[pl.ds(C, C)], ax_sh.at[pl.ds(base, C)], sr).wait()
        pltpu.make_async_copy(
            hbuf, xhat_h.at[pl.ds(hoff + base, C)], sx).wait()

    def scale2(src16, dst, j2):
        @plsc.parallel_loop(0, 2 * C // 16, unroll=2)
        def _scale(g):
            ch = 2 * j2 + g // 8
            wvv = ew_v[ch, pl.ds((g % 8) * 16, 16)]
            for jj in range(16):
                wrow = jnp.full((16,), wvv[jj])
                e = g * 16 + jj
                a0, a1 = plsc.unpack(src16[e, pl.ds(0, 32)], format=fmt)
                a2, a3 = plsc.unpack(src16[e, pl.ds(32, 32)], format=fmt)
                dst[e, pl.ds(0, 16)] = a0 * wrow
                dst[e, pl.ds(16, 16)] = a1 * wrow
                dst[e, pl.ds(32, 16)] = a2 * wrow
                dst[e, pl.ds(48, 16)] = a3 * wrow

    def start_gather2(j2, buf16):
        pltpu.async_copy(xhat_h.at[cols_v.at[2 * j2]],
                         buf16.at[pl.ds(0, C)], sg)
        pltpu.async_copy(xhat_h.at[cols_v.at[2 * j2 + 1]],
                         buf16.at[pl.ds(C, C)], sg)

    def wait_gather2(buf16):
        pltpu.make_async_copy(xhat_h.at[pl.ds(0, 2 * C)], buf16, sg).wait()

    def start_scatter2(j2, buf):
        pltpu.async_copy(buf.at[pl.ds(0, C)], ax_sh.at[rows_v.at[2 * j2]],
                         ss, add=True)
        pltpu.async_copy(buf.at[pl.ds(C, C)], ax_sh.at[rows_v.at[2 * j2 + 1]],
                         ss, add=True)

    def wait_scatter_any():
        pltpu.make_async_copy(gbuf, ax_sh.at[pl.ds(0, 2 * C)], ss).wait()

    NSC = NCHUNK // 2

    def edge_phase():
        start_gather2(0, gb16a)
        start_gather2(1, gb16b)
        wait_gather2(gb16a)
        scale2(gb16a, gbuf, 0)
        start_scatter2(0, gbuf)
        start_gather2(2, gb16a)
        wait_gather2(gb16b)
        scale2(gb16b, gbuf1, 1)
        start_scatter2(1, gbuf1)
        start_gather2(3, gb16b)

        @pl.loop(1, NSC // 2 - 1)
        def _pair(i):
            j0 = 2 * i
            wait_gather2(gb16a)
            wait_scatter_any()
            scale2(gb16a, gbuf, j0)
            start_scatter2(j0, gbuf)
            start_gather2(j0 + 2, gb16a)
            wait_gather2(gb16b)
            wait_scatter_any()
            scale2(gb16b, gbuf1, j0 + 1)
            start_scatter2(j0 + 1, gbuf1)
            start_gather2(j0 + 3, gb16b)

        wait_gather2(gb16a)
        wait_scatter_any()
        scale2(gb16a, gbuf, NSC - 2)
        start_scatter2(NSC - 2, gbuf)
        wait_gather2(gb16b)
        wait_scatter_any()
        scale2(gb16b, gbuf1, NSC - 1)
        start_scatter2(NSC - 1, gbuf1)
        wait_scatter_any()
        wait_scatter_any()

    act_phase()
    plsc.subcore_barrier()

    @pl.loop(0, ITERS - 1)
    def _iter(it):
        edge_phase()
        plsc.subcore_barrier()
        act_phase()
        plsc.subcore_barrier()

    pltpu.async_copy(xhat_h.at[oiv.at[pl.ds(s * 8, 8)]],
                     gb16a.at[pl.ds(0, 8)], sem).wait()
    wv16 = wov[pl.ds(s * 8, 16)]
    for j in range(8):
        wrow = jnp.full((16,), wv16[j])
        a0, a1 = plsc.unpack(gb16a[j, pl.ds(0, 32)], format=fmt)
        a2, a3 = plsc.unpack(gb16a[j, pl.ds(32, 32)], format=fmt)
        gbuf[j, pl.ds(0, 16)] = a0 * wrow
        gbuf[j, pl.ds(16, 16)] = a1 * wrow
        gbuf[j, pl.ds(32, 16)] = a2 * wrow
        gbuf[j, pl.ds(48, 16)] = a3 * wrow
    pltpu.sync_copy(gbuf.at[pl.ds(0, 8)], o_h.at[c, pl.ds(s * 8, 8)])


_sc_call = pl.kernel(
    _body,
    out_type=(
        jax.ShapeDtypeStruct((2, N_OUT, HALF), jnp.float32),
        jax.ShapeDtypeStruct((2 * N_PAD, HALF), jnp.bfloat16),
        jax.ShapeDtypeStruct((2 * N_PAD, HALF), jnp.float32),
    ),
    mesh=_mesh,
    compiler_params=pltpu.CompilerParams(use_tc_tiling_on_sc=False,
                                         needs_layout_passes=False),
    scratch_types=[
        pltpu.VMEM((NCHUNK, C), jnp.int32),
        pltpu.VMEM((NCHUNK, C), jnp.int32),
        pltpu.VMEM((NCHUNK, C), jnp.float32),
        pltpu.VMEM((2 * C, HALF), jnp.float32),
        pltpu.VMEM((2 * C, HALF), jnp.float32),
        pltpu.VMEM((2 * C, HALF), jnp.bfloat16),
        pltpu.VMEM((2 * C, HALF), jnp.bfloat16),
        pltpu.VMEM((C, HALF), jnp.bfloat16),
        pltpu.VMEM((N_IN,), jnp.float32),
        pltpu.VMEM((N_OUT + 16,), jnp.float32),
        pltpu.VMEM((N_IN,), jnp.int32),
        pltpu.VMEM((N_OUT,), jnp.int32),
        pltpu.VMEM((ROWS_PER_TILE,), jnp.float32),
        pltpu.SemaphoreType.DMA,
        pltpu.SemaphoreType.DMA,
        pltpu.SemaphoreType.DMA,
        pltpu.SemaphoreType.DMA,
        pltpu.SemaphoreType.DMA,
        pltpu.VMEM_SHARED((N_PAD, HALF), jnp.float32),
    ],
)


def kernel(x, w_in, edge_w, biases, w_out, rows, cols, in_idx, out_idx):
    xt2 = x.T.reshape(N_IN, 2, HALF).transpose(1, 0, 2)
    biases_p = jnp.pad(biases.reshape(-1), (0, N_PAD - N_NODES))
    n_pad_e = 16 * EDGES_PER_TILE - N_EDGES
    pad_idx = ((jnp.arange(n_pad_e, dtype=jnp.int32) * 37) % N_NODES)
    cols_p = jnp.concatenate([cols, pad_idx]).reshape(16, NCHUNK, C)
    rows_p = jnp.concatenate([rows, pad_idx]).reshape(16, NCHUNK, C)
    ew_p = jnp.concatenate(
        [edge_w, jnp.zeros((n_pad_e,), jnp.float32)]).reshape(16, NCHUNK, C)

    o, _, _ = _sc_call(xt2, w_in, biases_p, w_out,
                       in_idx.astype(jnp.int32), out_idx.astype(jnp.int32),
                       cols_p, rows_p, ew_p)
    return jnp.concatenate([o[0], o[1]], axis=1).T


# speedup vs baseline: 1.0741x; 1.0015x over previous
import jax
import jax.numpy as jnp
from jax import lax
from jax.experimental import pallas as pl
from jax.experimental.pallas import tpu as pltpu, tpu_sc as plsc

N_NODES = 10000
N_PAD = 10240
ROWS_PER_TILE = 640
N_EDGES = 160000
EDGES_PER_TILE = 10240
NCHUNK = 80
C = 128
N_IN = 128
N_OUT = 128
BATCH = 128
HALF = 64
ITERS = 20
LEAK = 0.01

_mesh = plsc.VectorSubcoreMesh(
    core_axis_name="c", subcore_axis_name="s", num_cores=2, num_subcores=16)

def _mml_vec(v):
    v = jnp.where(v < 0, v * LEAK, v)
    safe = jnp.where(v > 0.5, v, 1.0)
    return jnp.where(v > 0.5, 1.0 - 0.25 / safe, v)


def _body(xt_h, w_in_h, biases_h, w_out_h, in_idx_h, out_idx_h,
          cols_h, rows_h, ew_h,
          o_h, xhat_h, bin_h,
          cols_v, rows_v, ew_v, gbuf, gbuf1, gb16a, gb16b, hbuf,
          wiv, wov, iiv, oiv, bv, sem, sg, ss, sx, sr, ax_sh):
    fmt = plsc.PackFormat.INTERLEAVED
    c = lax.axis_index("c")
    s = lax.axis_index("s")
    base = s * ROWS_PER_TILE
    hoff = c * N_PAD

    pltpu.sync_copy(cols_h.at[s], cols_v)
    pltpu.sync_copy(rows_h.at[s], rows_v)
    pltpu.sync_copy(ew_h.at[s], ew_v)
    pltpu.sync_copy(biases_h.at[pl.ds(base, ROWS_PER_TILE)], bv)
    pltpu.sync_copy(out_idx_h, oiv)
    pltpu.sync_copy(w_out_h, wov.at[pl.ds(0, N_OUT)])

    @pl.loop(0, NCHUNK)
    def _shift(ch):
        for g in range(8):
            sl = pl.ds(g * 16, 16)
            cols_v[ch, sl] = cols_v[ch, sl] + hoff

    @pl.loop(0, N_OUT // 16)
    def _shift_o(g):
        sl = pl.ds(g * 16, 16)
        oiv[sl] = oiv[sl] + hoff

    @pl.loop(0, ROWS_PER_TILE // 16)
    def _bias_row(g):
        bvv = bv[pl.ds(g * 16, 16)]
        for j in range(16):
            row = jnp.full((16,), bvv[j])
            for k in range(4):
                gbuf1[j, pl.ds(k * 16, 16)] = row
        pltpu.sync_copy(gbuf1.at[pl.ds(0, 16)],
                        ax_sh.at[pl.ds(base + g * 16, 16)])

    plsc.subcore_barrier()

    @pl.when(s == 0)
    def _input_scatter():
        pltpu.sync_copy(w_in_h, wiv)
        pltpu.sync_copy(in_idx_h, iiv)
        pltpu.sync_copy(xt_h.at[c], gbuf.at[pl.ds(0, C)])

        @pl.loop(0, N_IN // 16)
        def _scale_in(g):
            wvv = wiv[pl.ds(g * 16, 16)]
            for j in range(16):
                wrow = jnp.full((16,), wvv[j])
                r = g * 16 + j
                for k in range(4):
                    sl = pl.ds(k * 16, 16)
                    gbuf[r, sl] = gbuf[r, sl] * wrow

        pltpu.sync_copy(gbuf.at[pl.ds(0, C)], ax_sh.at[iiv], add=True)

    plsc.subcore_barrier()

    @pl.loop(0, ROWS_PER_TILE // C)
    def _save_bin(i):
        off = base + i * C
        pltpu.sync_copy(ax_sh.at[pl.ds(off, C)], gbuf1.at[pl.ds(C, C)])
        pltpu.sync_copy(gbuf1.at[pl.ds(C, C)], bin_h.at[pl.ds(hoff + off, C)])

    def _mml_pack_buf():
        @plsc.parallel_loop(0, C, unroll=2)
        def _act_row(r):
            v0 = _mml_vec(gbuf1[r, pl.ds(0, 16)])
            v1 = _mml_vec(gbuf1[r, pl.ds(16, 16)])
            v2 = _mml_vec(gbuf1[r, pl.ds(32, 16)])
            v3 = _mml_vec(gbuf1[r, pl.ds(48, 16)])
            hbuf[r, pl.ds(0, 32)] = plsc.pack(v0, v1, format=fmt)
            hbuf[r, pl.ds(32, 32)] = plsc.pack(v2, v3, format=fmt)

    def act_phase():
        bin_cp = pltpu.async_copy(
            bin_h.at[pl.ds(hoff + base, C)], gbuf1.at[pl.ds(C, C)], sem)
        pltpu.sync_copy(ax_sh.at[pl.ds(base, C)], gbuf1.at[pl.ds(0, C)])
        _mml_pack_buf()
        pltpu.async_copy(hbuf, xhat_h.at[pl.ds(hoff + base, C)], sx)
        bin_cp.wait()
        pltpu.async_copy(gbuf1.at[pl.ds(C, C)], ax_sh.at[pl.ds(base, C)], sr)

        @pl.loop(1, ROWS_PER_TILE // C)
        def _act_sub(i):
            off = base + i * C
            pltpu.make_async_copy(
                gbuf1.at[pl.ds(C, C)], ax_sh.at[pl.ds(base, C)], sr).wait()
            bin_cp = pltpu.async_copy(
                bin_h.at[pl.ds(hoff + off, C)], gbuf1.at[pl.ds(C, C)], sem)
            pltpu.sync_copy(ax_sh.at[pl.ds(off, C)], gbuf1.at[pl.ds(0, C)])
            pltpu.make_async_copy(
                hbuf, xhat_h.at[pl.ds(hoff + base, C)], sx).wait()
            _mml_pack_buf()
            pltpu.async_copy(hbuf, xhat_h.at[pl.ds(hoff + off, C)], sx)
            bin_cp.wait()
            pltpu.async_copy(
                gbuf1.at[pl.ds(C, C)], ax_sh.at[pl.ds(off, C)], sr)

        pltpu.make_async_copy(
            gbuf1.at[pl.ds(C, C)], ax_sh.at[pl.ds(base, C)], sr).wait()
        pltpu.make_async_copy(
            hbuf, xhat_h.at[pl.ds(hoff + base, C)], sx).wait()

    def scale2(src16, dst, j2):
        @plsc.parallel_loop(0, 2 * C // 16, unroll=2)
        def _scale(g):
            ch = 2 * j2 + g // 8
            wvv = ew_v[ch, pl.ds((g % 8) * 16, 16)]
            for jj in range(16):
                wrow = jnp.full((16,), wvv[jj])
                e = g * 16 + jj
                a0, a1 = plsc.unpack(src16[e, pl.ds(0, 32)], format=fmt)
                a2, a3 = plsc.unpack(src16[e, pl.ds(32, 32)], format=fmt)
                dst[e, pl.ds(0, 16)] = a0 * wrow
                dst[e, pl.ds(16, 16)] = a1 * wrow
                dst[e, pl.ds(32, 16)] = a2 * wrow
                dst[e, pl.ds(48, 16)] = a3 * wrow

    def start_gather2(j2, buf16):
        pltpu.async_copy(xhat_h.at[cols_v.at[2 * j2]],
                         buf16.at[pl.ds(0, C)], sg)
        pltpu.async_copy(xhat_h.at[cols_v.at[2 * j2 + 1]],
                         buf16.at[pl.ds(C, C)], sg)

    def wait_gather2(buf16):
        pltpu.make_async_copy(xhat_h.at[pl.ds(0, 2 * C)], buf16, sg).wait()

    def start_scatter2(j2, buf):
        pltpu.async_copy(buf.at[pl.ds(0, C)], ax_sh.at[rows_v.at[2 * j2]],
                         ss, add=True)
        pltpu.async_copy(buf.at[pl.ds(C, C)], ax_sh.at[rows_v.at[2 * j2 + 1]],
                         ss, add=True)

    def wait_scatter_any():
        pltpu.make_async_copy(gbuf, ax_sh.at[pl.ds(0, 2 * C)], ss).wait()

    NSC = NCHUNK // 2

    def edge_phase():
        start_gather2(0, gb16a)
        start_gather2(1, gb16b)
        wait_gather2(gb16a)
        scale2(gb16a, gbuf, 0)
        start_scatter2(0, gbuf)
        start_gather2(2, gb16a)
        wait_gather2(gb16b)
        scale2(gb16b, gbuf1, 1)
        start_scatter2(1, gbuf1)
        start_gather2(3, gb16b)

        @pl.loop(1, NSC // 2 - 1)
        def _pair(i):
            j0 = 2 * i
            wait_gather2(gb16a)
            wait_scatter_any()
            scale2(gb16a, gbuf, j0)
            start_scatter2(j0, gbuf)
            start_gather2(j0 + 2, gb16a)
            wait_gather2(gb16b)
            wait_scatter_any()
            scale2(gb16b, gbuf1, j0 + 1)
            start_scatter2(j0 + 1, gbuf1)
            start_gather2(j0 + 3, gb16b)

        wait_gather2(gb16a)
        wait_scatter_any()
        scale2(gb16a, gbuf, NSC - 2)
        start_scatter2(NSC - 2, gbuf)
        wait_gather2(gb16b)
        wait_scatter_any()
        scale2(gb16b, gbuf1, NSC - 1)
        start_scatter2(NSC - 1, gbuf1)
        wait_scatter_any()
        wait_scatter_any()

    act_phase()
    plsc.subcore_barrier()

    @pl.loop(0, ITERS - 1)
    def _iter(it):
        edge_phase()
        plsc.subcore_barrier()
        act_phase()
        plsc.subcore_barrier()

    pltpu.async_copy(xhat_h.at[oiv.at[pl.ds(s * 8, 8)]],
                     gb16a.at[pl.ds(0, 8)], sem).wait()
    wv16 = wov[pl.ds(s * 8, 16)]
    for j in range(8):
        wrow = jnp.full((16,), wv16[j])
        a0, a1 = plsc.unpack(gb16a[j, pl.ds(0, 32)], format=fmt)
        a2, a3 = plsc.unpack(gb16a[j, pl.ds(32, 32)], format=fmt)
        gbuf[j, pl.ds(0, 16)] = a0 * wrow
        gbuf[j, pl.ds(16, 16)] = a1 * wrow
        gbuf[j, pl.ds(32, 16)] = a2 * wrow
        gbuf[j, pl.ds(48, 16)] = a3 * wrow
    pltpu.sync_copy(gbuf.at[pl.ds(0, 8)], o_h.at[c, pl.ds(s * 8, 8)])


_sc_call = pl.kernel(
    _body,
    out_type=(
        jax.ShapeDtypeStruct((2, N_OUT, HALF), jnp.float32),
        jax.ShapeDtypeStruct((2 * N_PAD, HALF), jnp.bfloat16),
        jax.ShapeDtypeStruct((2 * N_PAD, HALF), jnp.float32),
    ),
    mesh=_mesh,
    compiler_params=pltpu.CompilerParams(use_tc_tiling_on_sc=False,
                                         needs_layout_passes=False),
    scratch_types=[
        pltpu.VMEM((NCHUNK, C), jnp.int32),
        pltpu.VMEM((NCHUNK, C), jnp.int32),
        pltpu.VMEM((NCHUNK, C), jnp.float32),
        pltpu.VMEM((2 * C, HALF), jnp.float32),
        pltpu.VMEM((2 * C, HALF), jnp.float32),
        pltpu.VMEM((2 * C, HALF), jnp.bfloat16),
        pltpu.VMEM((2 * C, HALF), jnp.bfloat16),
        pltpu.VMEM((C, HALF), jnp.bfloat16),
        pltpu.VMEM((N_IN,), jnp.float32),
        pltpu.VMEM((N_OUT + 16,), jnp.float32),
        pltpu.VMEM((N_IN,), jnp.int32),
        pltpu.VMEM((N_OUT,), jnp.int32),
        pltpu.VMEM((ROWS_PER_TILE,), jnp.float32),
        pltpu.SemaphoreType.DMA,
        pltpu.SemaphoreType.DMA,
        pltpu.SemaphoreType.DMA,
        pltpu.SemaphoreType.DMA,
        pltpu.SemaphoreType.DMA,
        pltpu.VMEM_SHARED((N_PAD, HALF), jnp.float32),
    ],
)


def kernel(x, w_in, edge_w, biases, w_out, rows, cols, in_idx, out_idx):
    xt2 = x.T.reshape(N_IN, 2, HALF).transpose(1, 0, 2)
    biases_p = jnp.pad(biases.reshape(-1), (0, N_PAD - N_NODES))
    n_pad_e = 16 * EDGES_PER_TILE - N_EDGES
    pad_idx = ((jnp.arange(n_pad_e, dtype=jnp.int32) * 37) % N_NODES)
    cols_p = jnp.concatenate([cols, pad_idx]).reshape(16, NCHUNK, C)
    rows_p = jnp.concatenate([rows, pad_idx]).reshape(16, NCHUNK, C)
    ew_p = jnp.concatenate(
        [edge_w, jnp.zeros((n_pad_e,), jnp.float32)]).reshape(16, NCHUNK, C)

    o, _, _ = _sc_call(xt2, w_in, biases_p, w_out,
                       in_idx.astype(jnp.int32), out_idx.astype(jnp.int32),
                       cols_p, rows_p, ew_p)
    return jnp.concatenate([o[0], o[1]], axis=1).T
